# baseline (speedup 1.0000x reference)
import functools

import jax
import jax.numpy as jnp
from jax import lax
from jax.experimental import pallas as pl
from jax.experimental.pallas import tpu as pltpu
from jax.experimental.pallas import tpu_sc as plsc

D = 128
MINUTE_SIZE = 4
HOUR_SIZE = 24
WEEKDAY = 7
WD_PAD = 8
T_ROWS = MINUTE_SIZE * HOUR_SIZE
F_ROWS = T_ROWS * WD_PAD

NC, NS, L = 2, 16, 16
NW = NC * NS
CHUNK = 128
NBUF = 5
LA = 3


def _tc_prep(time, weekday, minute_embed, hour_embed, weekday_embed):
    def body(t_ref, wd_ref, m_ref, h_ref, w_ref, out_ref, idx_ref):
        r = lax.broadcasted_iota(jnp.int32, (F_ROWS, 1), 0)
        hour_id = r // (MINUTE_SIZE * WD_PAD)
        min_id = (r // WD_PAD) % MINUTE_SIZE
        wd_id = r % WD_PAD
        h_sel = jnp.zeros((F_ROWS, D), jnp.float32)
        for k in range(HOUR_SIZE):
            h_sel = jnp.where(hour_id == k, h_ref[k, :][None, :], h_sel)
        m_sel = jnp.zeros((F_ROWS, D), jnp.float32)
        for k in range(MINUTE_SIZE):
            m_sel = jnp.where(min_id == k, m_ref[k, :][None, :], m_sel)
        w_sel = jnp.zeros((F_ROWS, D), jnp.float32)
        for k in range(WEEKDAY):
            w_sel = jnp.where(wd_id == k, w_ref[k, :][None, :], w_sel)
        out_ref[...] = h_sel + m_sel + w_sel
        idx_ref[...] = t_ref[...] * WD_PAD + wd_ref[...]

    return pl.pallas_call(
        body,
        out_shape=(
            jax.ShapeDtypeStruct((F_ROWS, D), jnp.float32),
            jax.ShapeDtypeStruct(time.shape, jnp.int32),
        ),
    )(time, weekday, minute_embed, hour_embed, weekday_embed)


def _make_sc_gather(b_total):
    rows_pw = b_total // NW
    n_chunks = rows_pw // CHUNK
    assert n_chunks % NBUF == 0

    mesh = plsc.VectorSubcoreMesh(
        core_axis_name="c", subcore_axis_name="s", num_cores=NC, num_subcores=NS
    )

    @functools.partial(
        pl.kernel,
        out_type=jax.ShapeDtypeStruct((b_total, D), jnp.float32),
        mesh=mesh,
        scratch_types=[
            pltpu.VMEM_SHARED((F_ROWS, D), jnp.float32),
            pltpu.VMEM((rows_pw,), jnp.int32),
        ]
        + [pltpu.VMEM((CHUNK, D), jnp.float32)] * NBUF
        + [pltpu.SemaphoreType.DMA] * (2 * NBUF),
    )
    def sc_gather(table_hbm, idx_hbm, out_hbm, table_sh, idx_all,
                  rb0, rb1, rb2, rb3, rb4,
                  sg0, sg1, sg2, sg3, sg4, so0, so1, so2, so3, so4):
        rbufs = (rb0, rb1, rb2, rb3, rb4)
        sg = (sg0, sg1, sg2, sg3, sg4)
        so = (so0, so1, so2, so3, so4)
        cid = lax.axis_index("c")
        sid = lax.axis_index("s")
        wid = sid * NC + cid
        base = wid * rows_pw

        @pl.when(sid == 0)
        def _():
            pltpu.sync_copy(table_hbm, table_sh)

        pltpu.sync_copy(idx_hbm.at[pl.ds(base, rows_pw)], idx_all)

        plsc.subcore_barrier()

        def idx_ref(i):
            return idx_all.at[pl.ds(i * CHUNK, CHUNK)]

        def out_slice(i):
            return out_hbm.at[pl.ds(base + i * CHUNK, CHUNK)]

        for i in range(LA):
            pltpu.async_copy(table_sh.at[idx_ref(i)], rbufs[i], sg[i])

        def group(g, _):
            for b in range(NBUF):
                i = g * NBUF + b
                b2 = (b + LA) % NBUF

                @pl.when(i >= NBUF - LA)
                def _():
                    pltpu.make_async_copy(
                        rbufs[b2], out_slice(i - (NBUF - LA)), so[b2]
                    ).wait()

                @pl.when(i + LA < n_chunks)
                def _():
                    pltpu.async_copy(table_sh.at[idx_ref(i + LA)], rbufs[b2], sg[b2])

                pltpu.make_async_copy(table_sh.at[idx_ref(i)], rbufs[b], sg[b]).wait()
                pltpu.async_copy(rbufs[b], out_slice(i), so[b])
            return 0

        lax.fori_loop(0, n_chunks // NBUF, group, 0)

        for k in range(NBUF - LA, 0, -1):
            i = n_chunks - k
            pltpu.make_async_copy(rbufs[i % NBUF], out_slice(i), so[i % NBUF]).wait()

    return sc_gather


def kernel(time, weekday, minute_embed, hour_embed, weekday_embed):
    orig_shape = time.shape
    b_total = time.size
    table, fused_idx = _tc_prep(time, weekday, minute_embed, hour_embed, weekday_embed)
    out = _make_sc_gather(b_total)(table, fused_idx.reshape(-1))
    return out.reshape(*orig_shape, D)


# speedup vs baseline: 1.0031x; 1.0031x over previous
import functools

import jax
import jax.numpy as jnp
from jax import lax
from jax.experimental import pallas as pl
from jax.experimental.pallas import tpu as pltpu
from jax.experimental.pallas import tpu_sc as plsc

D = 128
MINUTE_SIZE = 4
HOUR_SIZE = 24
WEEKDAY = 7
WD_PAD = 8
T_ROWS = MINUTE_SIZE * HOUR_SIZE
F_ROWS = T_ROWS * WD_PAD

NC, NS, L = 2, 16, 16
NW = NC * NS
CHUNK = 128
NBUF = 5
LA = 2


def _tc_prep(time, weekday, minute_embed, hour_embed, weekday_embed):
    def body(t_ref, wd_ref, m_ref, h_ref, w_ref, out_ref, idx_ref):
        r = lax.broadcasted_iota(jnp.int32, (F_ROWS, 1), 0)
        hour_id = r // (MINUTE_SIZE * WD_PAD)
        min_id = (r // WD_PAD) % MINUTE_SIZE
        wd_id = r % WD_PAD
        h_sel = jnp.zeros((F_ROWS, D), jnp.float32)
        for k in range(HOUR_SIZE):
            h_sel = jnp.where(hour_id == k, h_ref[k, :][None, :], h_sel)
        m_sel = jnp.zeros((F_ROWS, D), jnp.float32)
        for k in range(MINUTE_SIZE):
            m_sel = jnp.where(min_id == k, m_ref[k, :][None, :], m_sel)
        w_sel = jnp.zeros((F_ROWS, D), jnp.float32)
        for k in range(WEEKDAY):
            w_sel = jnp.where(wd_id == k, w_ref[k, :][None, :], w_sel)
        out_ref[...] = h_sel + m_sel + w_sel
        idx_ref[...] = t_ref[...] * WD_PAD + wd_ref[...]

    return pl.pallas_call(
        body,
        out_shape=(
            jax.ShapeDtypeStruct((F_ROWS, D), jnp.float32),
            jax.ShapeDtypeStruct(time.shape, jnp.int32),
        ),
    )(time, weekday, minute_embed, hour_embed, weekday_embed)


def _make_sc_gather(b_total):
    rows_pw = b_total // NW
    n_chunks = rows_pw // CHUNK
    assert n_chunks % NBUF == 0

    mesh = plsc.VectorSubcoreMesh(
        core_axis_name="c", subcore_axis_name="s", num_cores=NC, num_subcores=NS
    )

    @functools.partial(
        pl.kernel,
        out_type=jax.ShapeDtypeStruct((b_total, D), jnp.float32),
        mesh=mesh,
        scratch_types=[
            pltpu.VMEM_SHARED((F_ROWS, D), jnp.float32),
            pltpu.VMEM((rows_pw,), jnp.int32),
        ]
        + [pltpu.VMEM((CHUNK, D), jnp.float32)] * NBUF
        + [pltpu.SemaphoreType.DMA] * (2 * NBUF),
    )
    def sc_gather(table_hbm, idx_hbm, out_hbm, table_sh, idx_all,
                  rb0, rb1, rb2, rb3, rb4,
                  sg0, sg1, sg2, sg3, sg4, so0, so1, so2, so3, so4):
        rbufs = (rb0, rb1, rb2, rb3, rb4)
        sg = (sg0, sg1, sg2, sg3, sg4)
        so = (so0, so1, so2, so3, so4)
        cid = lax.axis_index("c")
        sid = lax.axis_index("s")
        wid = sid * NC + cid
        base = wid * rows_pw

        @pl.when(sid == 0)
        def _():
            pltpu.sync_copy(table_hbm, table_sh)

        pltpu.sync_copy(idx_hbm.at[pl.ds(base, rows_pw)], idx_all)

        plsc.subcore_barrier()

        def idx_ref(i):
            return idx_all.at[pl.ds(i * CHUNK, CHUNK)]

        def out_slice(i):
            return out_hbm.at[pl.ds(base + i * CHUNK, CHUNK)]

        for i in range(LA):
            pltpu.async_copy(table_sh.at[idx_ref(i)], rbufs[i], sg[i])

        def group(g, _):
            for b in range(NBUF):
                i = g * NBUF + b
                b2 = (b + LA) % NBUF

                @pl.when(i >= NBUF - LA)
                def _():
                    pltpu.make_async_copy(
                        rbufs[b2], out_slice(i - (NBUF - LA)), so[b2]
                    ).wait()

                @pl.when(i + LA < n_chunks)
                def _():
                    pltpu.async_copy(table_sh.at[idx_ref(i + LA)], rbufs[b2], sg[b2])

                pltpu.make_async_copy(table_sh.at[idx_ref(i)], rbufs[b], sg[b]).wait()
                pltpu.async_copy(rbufs[b], out_slice(i), so[b])
            return 0

        lax.fori_loop(0, n_chunks // NBUF, group, 0)

        for k in range(NBUF - LA, 0, -1):
            i = n_chunks - k
            pltpu.make_async_copy(rbufs[i % NBUF], out_slice(i), so[i % NBUF]).wait()

    return sc_gather


def kernel(time, weekday, minute_embed, hour_embed, weekday_embed):
    orig_shape = time.shape
    b_total = time.size
    table, fused_idx = _tc_prep(time, weekday, minute_embed, hour_embed, weekday_embed)
    out = _make_sc_gather(b_total)(table, fused_idx.reshape(-1))
    return out.reshape(*orig_shape, D)


# speedup vs baseline: 1.0145x; 1.0113x over previous
import functools

import jax
import jax.numpy as jnp
from jax import lax
from jax.experimental import pallas as pl
from jax.experimental.pallas import tpu as pltpu
from jax.experimental.pallas import tpu_sc as plsc

D = 128
MINUTE_SIZE = 4
HOUR_SIZE = 24
WEEKDAY = 7
WD_PAD = 8
T_ROWS = MINUTE_SIZE * HOUR_SIZE
F_ROWS = T_ROWS * WD_PAD

NC, NS, L = 2, 16, 16
NW = NC * NS
CHUNK = 64
NBUF = 5
LA = 2


def _tc_prep(time, weekday, minute_embed, hour_embed, weekday_embed):
    def body(t_ref, wd_ref, m_ref, h_ref, w_ref, out_ref, idx_ref):
        r = lax.broadcasted_iota(jnp.int32, (F_ROWS, 1), 0)
        hour_id = r // (MINUTE_SIZE * WD_PAD)
        min_id = (r // WD_PAD) % MINUTE_SIZE
        wd_id = r % WD_PAD
        h_sel = jnp.zeros((F_ROWS, D), jnp.float32)
        for k in range(HOUR_SIZE):
            h_sel = jnp.where(hour_id == k, h_ref[k, :][None, :], h_sel)
        m_sel = jnp.zeros((F_ROWS, D), jnp.float32)
        for k in range(MINUTE_SIZE):
            m_sel = jnp.where(min_id == k, m_ref[k, :][None, :], m_sel)
        w_sel = jnp.zeros((F_ROWS, D), jnp.float32)
        for k in range(WEEKDAY):
            w_sel = jnp.where(wd_id == k, w_ref[k, :][None, :], w_sel)
        out_ref[...] = h_sel + m_sel + w_sel
        idx_ref[...] = t_ref[...] * WD_PAD + wd_ref[...]

    return pl.pallas_call(
        body,
        out_shape=(
            jax.ShapeDtypeStruct((F_ROWS, D), jnp.float32),
            jax.ShapeDtypeStruct(time.shape, jnp.int32),
        ),
    )(time, weekday, minute_embed, hour_embed, weekday_embed)


def _make_sc_gather(b_total):
    rows_pw = b_total // NW
    n_chunks = rows_pw // CHUNK
    assert n_chunks % NBUF == 0

    mesh = plsc.VectorSubcoreMesh(
        core_axis_name="c", subcore_axis_name="s", num_cores=NC, num_subcores=NS
    )

    @functools.partial(
        pl.kernel,
        out_type=jax.ShapeDtypeStruct((b_total, D), jnp.float32),
        mesh=mesh,
        scratch_types=[
            pltpu.VMEM_SHARED((F_ROWS, D), jnp.float32),
            pltpu.VMEM((rows_pw,), jnp.int32),
        ]
        + [pltpu.VMEM((CHUNK, D), jnp.float32)] * NBUF
        + [pltpu.SemaphoreType.DMA] * (2 * NBUF),
    )
    def sc_gather(table_hbm, idx_hbm, out_hbm, table_sh, idx_all, *scratch):
        rbufs = scratch[:NBUF]
        sg = scratch[NBUF:2 * NBUF]
        so = scratch[2 * NBUF:3 * NBUF]
        cid = lax.axis_index("c")
        sid = lax.axis_index("s")
        wid = sid * NC + cid
        base = wid * rows_pw

        @pl.when(sid == 0)
        def _():
            pltpu.sync_copy(table_hbm, table_sh)

        pltpu.sync_copy(idx_hbm.at[pl.ds(base, rows_pw)], idx_all)

        plsc.subcore_barrier()

        def idx_ref(i):
            return idx_all.at[pl.ds(i * CHUNK, CHUNK)]

        def out_slice(i):
            return out_hbm.at[pl.ds(base + i * CHUNK, CHUNK)]

        for i in range(LA):
            pltpu.async_copy(table_sh.at[idx_ref(i)], rbufs[i], sg[i])

        def group(g, _):
            for b in range(NBUF):
                i = g * NBUF + b
                b2 = (b + LA) % NBUF

                @pl.when(i >= NBUF - LA)
                def _():
                    pltpu.make_async_copy(
                        rbufs[b2], out_slice(i - (NBUF - LA)), so[b2]
                    ).wait()

                @pl.when(i + LA < n_chunks)
                def _():
                    pltpu.async_copy(table_sh.at[idx_ref(i + LA)], rbufs[b2], sg[b2])

                pltpu.make_async_copy(table_sh.at[idx_ref(i)], rbufs[b], sg[b]).wait()
                pltpu.async_copy(rbufs[b], out_slice(i), so[b])
            return 0

        lax.fori_loop(0, n_chunks // NBUF, group, 0)

        for k in range(NBUF - LA, 0, -1):
            i = n_chunks - k
            pltpu.make_async_copy(rbufs[i % NBUF], out_slice(i), so[i % NBUF]).wait()

    return sc_gather


def kernel(time, weekday, minute_embed, hour_embed, weekday_embed):
    orig_shape = time.shape
    b_total = time.size
    table, fused_idx = _tc_prep(time, weekday, minute_embed, hour_embed, weekday_embed)
    out = _make_sc_gather(b_total)(table, fused_idx.reshape(-1))
    return out.reshape(*orig_shape, D)
